# baseline (speedup 1.0000x reference)
import functools
import jax
import jax.numpy as jnp
from jax import lax
from jax.experimental import pallas as pl
from jax.experimental.pallas import tpu as pltpu
from jax.experimental.pallas import tpu_sc as plsc

M = 16384
N = 4096
C = 128
CS = 64
BQ = 256
NBLK = M // BQ

NW = 32
QW = M // NW
QB = 128
NCHUNK = QW // QB
BL = 1024


def _knn_block(pos_skip_ref, posT_ref, idx_ref, w_ref):
    q = pos_skip_ref[...]
    p = posT_ref[...]
    qsq = jnp.sum(q * q, axis=1, keepdims=True)
    psq = jnp.sum(p * p, axis=0, keepdims=True)
    cross = jax.lax.dot_general(
        q, p, (((1,), (0,)), ((), ())),
        preferred_element_type=jnp.float32)
    d2 = (qsq + psq) - (cross + cross)

    lane = jax.lax.broadcasted_iota(jnp.int32, (BQ, N), 1)
    inf = jnp.float32(jnp.inf)
    for k in range(3):
        m = jnp.min(d2, axis=1, keepdims=True)
        hit = d2 == m
        i = jnp.min(jnp.where(hit, lane, N), axis=1, keepdims=True)
        idx_ref[:, k:k+1] = i
        w_ref[:, k:k+1] = 1.0 / jnp.maximum(m, 1e-16)
        d2 = jnp.where(lane == i, inf, d2)


def _sc_interp(x_hbm, idx0_hbm, idx1_hbm, idx2_hbm, w0_hbm, w1_hbm, w2_hbm,
               out_hbm,
               idxk0, idxk1, idxk2, wk0, wk1, wk2,
               rows0, rows1, rows2, outv,
               sem0, sem1, sem2):
    wid = lax.axis_index("s") * 2 + lax.axis_index("c")
    base = wid * QW
    rows = (rows0, rows1, rows2)
    idxk = (idxk0, idxk1, idxk2)
    wk = (wk0, wk1, wk2)
    sems = (sem0, sem1, sem2)
    giota = lax.iota(jnp.int32, 16)

    def chunk(ci, carry):
        qoff = base + ci * QB
        for k, (ih, wh) in enumerate(((idx0_hbm, w0_hbm), (idx1_hbm, w1_hbm),
                                      (idx2_hbm, w2_hbm))):
            pltpu.sync_copy(ih.at[pl.ds(qoff, QB)], idxk[k])
            pltpu.sync_copy(wh.at[pl.ds(qoff, QB)], wk[k])
        copies = [pltpu.async_copy(x_hbm.at[idxk[k]], rows[k], sems[k])
                  for k in range(3)]
        for cp in copies:
            cp.wait()
        def gbody(g, c):
            w0 = wk0[pl.ds(g * 16, 16)]
            w1 = wk1[pl.ds(g * 16, 16)]
            w2 = wk2[pl.ds(g * 16, 16)]
            den = w0 + w1 + w2
            wn0 = w0 / den
            wn1 = w1 / den
            wn2 = w2 / den
            for ql in range(16):
                qa = g * 16 + ql
                lane = jnp.full((16,), ql, jnp.int32)
                s0 = jnp.take(wn0, lane)
                s1 = jnp.take(wn1, lane)
                s2 = jnp.take(wn2, lane)
                for cb in range(C // 16):
                    r0 = rows0[qa, pl.ds(cb * 16, 16)]
                    r1 = rows1[qa, pl.ds(cb * 16, 16)]
                    r2 = rows2[qa, pl.ds(cb * 16, 16)]
                    outv[qa, pl.ds(cb * 16, 16)] = s0 * r0 + s1 * r1 + s2 * r2
            return c

        lax.fori_loop(0, QB // 16, gbody, 0)
        pltpu.sync_copy(outv, out_hbm.at[pl.ds(qoff, QB)])
        return carry

    lax.fori_loop(0, NCHUNK, chunk, 0)


def _linear_block(interp_ref, x_skip_ref, w_ref, b_ref, out_ref):
    h = jnp.concatenate([interp_ref[...], x_skip_ref[...]], axis=1)
    out_ref[...] = jax.lax.dot_general(
        h, w_ref[...], (((1,), (1,)), ((), ())),
        preferred_element_type=jnp.float32) + b_ref[...]


def kernel(x, pos, batch, x_skip, pos_skip, batch_skip, W, b):
    posT = pos.T.astype(jnp.bfloat16).astype(jnp.float32)
    ps_r = pos_skip.astype(jnp.bfloat16).astype(jnp.float32)
    b2 = b.reshape(1, C)

    idx3, w3 = pl.pallas_call(
        _knn_block,
        grid=(NBLK,),
        in_specs=[
            pl.BlockSpec((BQ, 3), lambda i: (i, 0)),
            pl.BlockSpec((3, N), lambda i: (0, 0)),
        ],
        out_specs=[
            pl.BlockSpec((BQ, 3), lambda i: (i, 0)),
            pl.BlockSpec((BQ, 3), lambda i: (i, 0)),
        ],
        out_shape=[
            jax.ShapeDtypeStruct((M, 3), jnp.int32),
            jax.ShapeDtypeStruct((M, 3), jnp.float32),
        ],
    )(ps_r, posT)

    sc = functools.partial(
        pl.kernel,
        out_type=jax.ShapeDtypeStruct((M, C), jnp.float32),
        mesh=plsc.VectorSubcoreMesh(core_axis_name="c", subcore_axis_name="s"),
        scratch_types=[
            pltpu.VMEM((QB,), jnp.int32),
            pltpu.VMEM((QB,), jnp.int32),
            pltpu.VMEM((QB,), jnp.int32),
            pltpu.VMEM((QB,), jnp.float32),
            pltpu.VMEM((QB,), jnp.float32),
            pltpu.VMEM((QB,), jnp.float32),
            pltpu.VMEM((QB, C), jnp.float32),
            pltpu.VMEM((QB, C), jnp.float32),
            pltpu.VMEM((QB, C), jnp.float32),
            pltpu.VMEM((QB, C), jnp.float32),
            pltpu.SemaphoreType.DMA,
            pltpu.SemaphoreType.DMA,
            pltpu.SemaphoreType.DMA,
        ],
    )(_sc_interp)
    interp = sc(x, idx3[:, 0], idx3[:, 1], idx3[:, 2],
                w3[:, 0], w3[:, 1], w3[:, 2])

    out = pl.pallas_call(
        _linear_block,
        grid=(M // BL,),
        in_specs=[
            pl.BlockSpec((BL, C), lambda i: (i, 0)),
            pl.BlockSpec((BL, CS), lambda i: (i, 0)),
            pl.BlockSpec((C, C + CS), lambda i: (0, 0)),
            pl.BlockSpec((1, C), lambda i: (0, 0)),
        ],
        out_specs=pl.BlockSpec((BL, C), lambda i: (i, 0)),
        out_shape=jax.ShapeDtypeStruct((M, C), jnp.float32),
    )(interp, x_skip, W, b2)

    return (out, pos_skip, batch_skip)


# speedup vs baseline: 1.0233x; 1.0233x over previous
import functools
import jax
import jax.numpy as jnp
from jax import lax
from jax.experimental import pallas as pl
from jax.experimental.pallas import tpu as pltpu
from jax.experimental.pallas import tpu_sc as plsc

M = 16384
N = 4096
C = 128
CS = 64
BQ = 256
NBLK = M // BQ

NW = 32
QW = M // NW
QB = 64
NCHUNK = QW // QB


def _knn_block(pos_skip_ref, x_skip_ref, posT_ref, x_ref, w1_ref, w2_ref,
               b_ref, idx_ref, wn_ref, base_ref, xw_ref):
    @pl.when(pl.program_id(0) == 0)
    def _():
        xw_ref[...] = jax.lax.dot_general(
            x_ref[...], w1_ref[...], (((1,), (1,)), ((), ())),
            preferred_element_type=jnp.float32,
            precision=jax.lax.Precision.HIGHEST)

    q = pos_skip_ref[...]
    p = posT_ref[...]
    qsq = jnp.sum(q * q, axis=1, keepdims=True)
    psq = jnp.sum(p * p, axis=0, keepdims=True)
    cross = jax.lax.dot_general(
        q, p, (((1,), (0,)), ((), ())),
        preferred_element_type=jnp.float32)
    d2 = (qsq + psq) - (cross + cross)

    lane = jax.lax.broadcasted_iota(jnp.int32, (BQ, N), 1)
    inf = jnp.float32(jnp.inf)
    ws = []
    for k in range(3):
        m = jnp.min(d2, axis=1, keepdims=True)
        hit = d2 == m
        i = jnp.min(jnp.where(hit, lane, N), axis=1, keepdims=True)
        idx_ref[:, k:k+1] = i
        ws.append(1.0 / jnp.maximum(m, 1e-16))
        d2 = jnp.where(hit, inf, d2)
    den = ws[0] + ws[1] + ws[2]
    for k in range(3):
        wn_ref[:, k:k+1] = ws[k] / den

    base_ref[...] = jax.lax.dot_general(
        x_skip_ref[...], w2_ref[...], (((1,), (1,)), ((), ())),
        preferred_element_type=jnp.float32,
        precision=jax.lax.Precision.HIGHEST) + b_ref[...]


def _sc_combine(xw_hbm, base_hbm, idx0_hbm, idx1_hbm, idx2_hbm,
                w0_hbm, w1_hbm, w2_hbm, out_hbm,
                idxkA, idxkB, wkA, wkB, rowsA, rowsB, bbA, bbB, outv,
                gsemA, gsemB, bsemA, bsemB):
    wid = lax.axis_index("s") * 2 + lax.axis_index("c")
    base = wid * QW
    idxk = (idxkA, idxkB)
    wk = (wkA, wkB)
    rows = (rowsA, rowsB)
    bb = (bbA, bbB)
    gsem = (gsemA, gsemB)
    bsem = (bsemA, bsemB)
    ih = (idx0_hbm, idx1_hbm, idx2_hbm)
    wh = (w0_hbm, w1_hbm, w2_hbm)

    def stage(ci):
        s = ci & 1
        qoff = base + ci * QB
        for k in range(3):
            pltpu.sync_copy(ih[k].at[pl.ds(qoff, QB)], idxk[s][k])
            pltpu.sync_copy(wh[k].at[pl.ds(qoff, QB)], wk[s][k])
        cps = [pltpu.async_copy(xw_hbm.at[idxk[s][k]], rows[s][k], gsem[s])
               for k in range(3)]
        cps.append(pltpu.async_copy(base_hbm.at[pl.ds(qoff, QB)], bb[s],
                                    bsem[s]))
        return cps

    pending = {0: stage(0)}
    for ci in range(NCHUNK):
        if ci + 1 < NCHUNK:
            pending[ci + 1] = stage(ci + 1)
        for cp in pending.pop(ci):
            cp.wait()
        s = ci & 1
        r0, r1, r2 = rows[s]
        w0r, w1r, w2r = wk[s]
        bbr = bb[s]

        def qbody(qa, c, r0=r0, r1=r1, r2=r2, w0r=w0r, w1r=w1r, w2r=w2r,
                  bbr=bbr):
            g16 = (qa // 16) * 16
            ql = qa - g16
            lane = jnp.full((16,), ql, jnp.int32)
            s0 = jnp.take(w0r[pl.ds(g16, 16)], lane)
            s1 = jnp.take(w1r[pl.ds(g16, 16)], lane)
            s2 = jnp.take(w2r[pl.ds(g16, 16)], lane)
            for cb in range(C // 16):
                sl = pl.ds(cb * 16, 16)
                outv[qa, sl] = (bbr[qa, sl] + s0 * r0[qa, sl]
                                + s1 * r1[qa, sl] + s2 * r2[qa, sl])
            return c

        lax.fori_loop(0, QB, qbody, 0)
        pltpu.sync_copy(outv, out_hbm.at[pl.ds(base + ci * QB, QB)])


def kernel(x, pos, batch, x_skip, pos_skip, batch_skip, W, b):
    posT = pos.T.astype(jnp.bfloat16).astype(jnp.float32)
    ps_r = pos_skip.astype(jnp.bfloat16).astype(jnp.float32)
    b2 = b.reshape(1, C)

    idx3, wn3, base, xw = pl.pallas_call(
        _knn_block,
        grid=(NBLK,),
        in_specs=[
            pl.BlockSpec((BQ, 3), lambda i: (i, 0)),
            pl.BlockSpec((BQ, CS), lambda i: (i, 0)),
            pl.BlockSpec((3, N), lambda i: (0, 0)),
            pl.BlockSpec((N, C), lambda i: (0, 0)),
            pl.BlockSpec((C, C), lambda i: (0, 0)),
            pl.BlockSpec((C, CS), lambda i: (0, 0)),
            pl.BlockSpec((1, C), lambda i: (0, 0)),
        ],
        out_specs=[
            pl.BlockSpec((BQ, 3), lambda i: (i, 0)),
            pl.BlockSpec((BQ, 3), lambda i: (i, 0)),
            pl.BlockSpec((BQ, C), lambda i: (i, 0)),
            pl.BlockSpec((N, C), lambda i: (0, 0)),
        ],
        out_shape=[
            jax.ShapeDtypeStruct((M, 3), jnp.int32),
            jax.ShapeDtypeStruct((M, 3), jnp.float32),
            jax.ShapeDtypeStruct((M, C), jnp.float32),
            jax.ShapeDtypeStruct((N, C), jnp.float32),
        ],
    )(ps_r, x_skip, posT, x, W[:, :C], W[:, C:], b2)

    sc = functools.partial(
        pl.kernel,
        out_type=jax.ShapeDtypeStruct((M, C), jnp.float32),
        mesh=plsc.VectorSubcoreMesh(core_axis_name="c", subcore_axis_name="s"),
        scratch_types=[
            tuple(pltpu.VMEM((QB,), jnp.int32) for _ in range(3)),
            tuple(pltpu.VMEM((QB,), jnp.int32) for _ in range(3)),
            tuple(pltpu.VMEM((QB,), jnp.float32) for _ in range(3)),
            tuple(pltpu.VMEM((QB,), jnp.float32) for _ in range(3)),
            tuple(pltpu.VMEM((QB, C), jnp.float32) for _ in range(3)),
            tuple(pltpu.VMEM((QB, C), jnp.float32) for _ in range(3)),
            pltpu.VMEM((QB, C), jnp.float32),
            pltpu.VMEM((QB, C), jnp.float32),
            pltpu.VMEM((QB, C), jnp.float32),
            pltpu.SemaphoreType.DMA,
            pltpu.SemaphoreType.DMA,
            pltpu.SemaphoreType.DMA,
            pltpu.SemaphoreType.DMA,
        ],
    )(_sc_combine)
    out = sc(xw, base, idx3[:, 0], idx3[:, 1], idx3[:, 2],
             wn3[:, 0], wn3[:, 1], wn3[:, 2])

    return (out, pos_skip, batch_skip)


# speedup vs baseline: 1.0522x; 1.0282x over previous
import functools
import jax
import jax.numpy as jnp
from jax import lax
from jax.experimental import pallas as pl
from jax.experimental.pallas import tpu as pltpu
from jax.experimental.pallas import tpu_sc as plsc

M = 16384
N = 4096
C = 128
CS = 64
BQ = 256
NBLK = M // BQ

NW = 32
QW = M // NW
QB = 64
NCHUNK = QW // QB


def _knn_block(pos_skip_ref, x_skip_ref, posT_ref, x_ref, w1_ref, w2_ref,
               b_ref, i0_ref, i1_ref, i2_ref, n0_ref, n1_ref, n2_ref,
               base_ref, xw_ref):
    @pl.when(pl.program_id(0) == 0)
    def _():
        xw_ref[...] = jax.lax.dot_general(
            x_ref[...], w1_ref[...], (((1,), (1,)), ((), ())),
            preferred_element_type=jnp.float32,
            precision=jax.lax.Precision.HIGHEST)

    q = pos_skip_ref[...]
    p = posT_ref[...]
    qsq = jnp.sum(q * q, axis=1, keepdims=True)
    psq = jnp.sum(p * p, axis=0, keepdims=True)
    cross = jax.lax.dot_general(
        q, p, (((1,), (0,)), ((), ())),
        preferred_element_type=jnp.float32)
    d2 = (qsq + psq) - (cross + cross)

    lane = jax.lax.broadcasted_iota(jnp.int32, (BQ, N), 1)
    inf = jnp.float32(jnp.inf)
    ws = []
    irefs = (i0_ref, i1_ref, i2_ref)
    nrefs = (n0_ref, n1_ref, n2_ref)
    for k in range(3):
        m = jnp.min(d2, axis=1, keepdims=True)
        hit = d2 == m
        i = jnp.min(jnp.where(hit, lane, N), axis=1, keepdims=True)
        irefs[k][...] = i
        ws.append(1.0 / jnp.maximum(m, 1e-16))
        d2 = jnp.where(hit, inf, d2)
    den = ws[0] + ws[1] + ws[2]
    for k in range(3):
        nrefs[k][...] = ws[k] / den

    base_ref[...] = jax.lax.dot_general(
        x_skip_ref[...], w2_ref[...], (((1,), (1,)), ((), ())),
        preferred_element_type=jnp.float32,
        precision=jax.lax.Precision.HIGHEST) + b_ref[...]


def _sc_combine(xw_hbm, base_hbm, idx0_hbm, idx1_hbm, idx2_hbm,
                w0_hbm, w1_hbm, w2_hbm, out_hbm,
                ib0, ib1, ib2, wb0, wb1, wb2,
                rowsA, rowsB, bbA, bbB, outv,
                gsemA, gsemB, bsemA, bsemB):
    wid = lax.axis_index("s") * 2 + lax.axis_index("c")
    base = wid * QW
    rows = (rowsA, rowsB)
    bb = (bbA, bbB)
    gsem = (gsemA, gsemB)
    bsem = (bsemA, bsemB)
    ibig = (ib0, ib1, ib2)
    wbig = (wb0, wb1, wb2)

    for ih, dst in ((idx0_hbm, ib0), (idx1_hbm, ib1), (idx2_hbm, ib2),
                    (w0_hbm, wb0), (w1_hbm, wb1), (w2_hbm, wb2)):
        pltpu.sync_copy(ih.at[pl.ds(base, QW)], dst)

    def stage(ci):
        s = ci & 1
        qoff = base + ci * QB
        cps = [pltpu.async_copy(
                   xw_hbm.at[ibig[k].at[pl.ds(ci * QB, QB)]],
                   rows[s][k], gsem[s])
               for k in range(3)]
        cps.append(pltpu.async_copy(base_hbm.at[pl.ds(qoff, QB)], bb[s],
                                    bsem[s]))
        return cps

    pending = {0: stage(0)}
    for ci in range(NCHUNK):
        if ci + 1 < NCHUNK:
            pending[ci + 1] = stage(ci + 1)
        for cp in pending.pop(ci):
            cp.wait()
        s = ci & 1
        r0, r1, r2 = rows[s]
        bbr = bb[s]
        coff = ci * QB

        def qbody(qa, c, r0=r0, r1=r1, r2=r2, bbr=bbr, coff=coff):
            g16 = (qa // 16) * 16
            ql = qa - g16
            lane = jnp.full((16,), ql, jnp.int32)
            s0 = jnp.take(wb0[pl.ds(coff + g16, 16)], lane)
            s1 = jnp.take(wb1[pl.ds(coff + g16, 16)], lane)
            s2 = jnp.take(wb2[pl.ds(coff + g16, 16)], lane)
            for cb in range(C // 16):
                sl = pl.ds(cb * 16, 16)
                outv[qa, sl] = (bbr[qa, sl] + s0 * r0[qa, sl]
                                + s1 * r1[qa, sl] + s2 * r2[qa, sl])
            return c

        lax.fori_loop(0, QB, qbody, 0)
        pltpu.sync_copy(outv, out_hbm.at[pl.ds(base + ci * QB, QB)])


def kernel(x, pos, batch, x_skip, pos_skip, batch_skip, W, b):
    posT = pos.T.astype(jnp.bfloat16).astype(jnp.float32)
    ps_r = pos_skip.astype(jnp.bfloat16).astype(jnp.float32)
    b2 = b.reshape(1, C)

    i0, i1, i2, n0, n1, n2, base, xw = pl.pallas_call(
        _knn_block,
        grid=(NBLK,),
        in_specs=[
            pl.BlockSpec((BQ, 3), lambda i: (i, 0)),
            pl.BlockSpec((BQ, CS), lambda i: (i, 0)),
            pl.BlockSpec((3, N), lambda i: (0, 0)),
            pl.BlockSpec((N, C), lambda i: (0, 0)),
            pl.BlockSpec((C, C), lambda i: (0, 0)),
            pl.BlockSpec((C, CS), lambda i: (0, 0)),
            pl.BlockSpec((1, C), lambda i: (0, 0)),
        ],
        out_specs=(
            [pl.BlockSpec((BQ, 1), lambda i: (i, 0)) for _ in range(6)]
            + [pl.BlockSpec((BQ, C), lambda i: (i, 0)),
               pl.BlockSpec((N, C), lambda i: (0, 0))]
        ),
        out_shape=(
            [jax.ShapeDtypeStruct((M, 1), jnp.int32) for _ in range(3)]
            + [jax.ShapeDtypeStruct((M, 1), jnp.float32) for _ in range(3)]
            + [jax.ShapeDtypeStruct((M, C), jnp.float32),
               jax.ShapeDtypeStruct((N, C), jnp.float32)]
        ),
    )(ps_r, x_skip, posT, x, W[:, :C], W[:, C:], b2)

    sc = functools.partial(
        pl.kernel,
        out_type=jax.ShapeDtypeStruct((M, C), jnp.float32),
        mesh=plsc.VectorSubcoreMesh(core_axis_name="c", subcore_axis_name="s"),
        scratch_types=[
            pltpu.VMEM((QW,), jnp.int32),
            pltpu.VMEM((QW,), jnp.int32),
            pltpu.VMEM((QW,), jnp.int32),
            pltpu.VMEM((QW,), jnp.float32),
            pltpu.VMEM((QW,), jnp.float32),
            pltpu.VMEM((QW,), jnp.float32),
            tuple(pltpu.VMEM((QB, C), jnp.float32) for _ in range(3)),
            tuple(pltpu.VMEM((QB, C), jnp.float32) for _ in range(3)),
            pltpu.VMEM((QB, C), jnp.float32),
            pltpu.VMEM((QB, C), jnp.float32),
            pltpu.VMEM((QB, C), jnp.float32),
            pltpu.SemaphoreType.DMA,
            pltpu.SemaphoreType.DMA,
            pltpu.SemaphoreType.DMA,
            pltpu.SemaphoreType.DMA,
        ],
    )(_sc_combine)
    out = sc(xw, base, i0.reshape(M), i1.reshape(M), i2.reshape(M),
             n0.reshape(M), n1.reshape(M), n2.reshape(M))

    return (out, pos_skip, batch_skip)
